# baseline (speedup 1.0000x reference)
import functools

import jax
import jax.numpy as jnp
from jax import lax
from jax.experimental import pallas as pl
from jax.experimental.pallas import tpu as pltpu
from jax.experimental.pallas import tpu_sc as plsc

NC = 2
NS = 16
NW = NC * NS
CHUNK = 128
EPS = 1e-5


def _sc_mesh():
    return plsc.VectorSubcoreMesh(core_axis_name="c", subcore_axis_name="s",
                                  num_cores=NC, num_subcores=NS)


def _make_deg(e_pad, n_pad):
    steps = e_pad // (NW * CHUNK)
    stripe = n_pad // NS

    @functools.partial(
        pl.kernel,
        out_type=jax.ShapeDtypeStruct((NC * n_pad,), jnp.float32),
        mesh=_sc_mesh(),
        scratch_types=[
            pltpu.VMEM((CHUNK,), jnp.int32),
            pltpu.VMEM((CHUNK,), jnp.int32),
            pltpu.VMEM((CHUNK,), jnp.float32),
            pltpu.VMEM_SHARED((n_pad,), jnp.float32),
            pltpu.SemaphoreType.DMA,
            pltpu.SemaphoreType.DMA,
        ],
    )
    def deg_kernel(dst_hbm, out_hbm, didx_a, didx_b, ones, acc, sem_a, sem_b):
        c = lax.axis_index("c")
        s = lax.axis_index("s")
        wid = s * NC + c

        one16 = jnp.ones((16,), jnp.float32)
        zero16 = jnp.zeros((16,), jnp.float32)

        def _fill_zero(i, carry):
            ones[pl.ds(i * 16, 16)] = zero16
            return carry

        lax.fori_loop(0, CHUNK // 16, _fill_zero, 0)

        def _zacc(i, carry):
            pltpu.sync_copy(ones, acc.at[pl.ds(s * stripe + i * CHUNK, CHUNK)])
            return carry

        lax.fori_loop(0, stripe // CHUNK, _zacc, 0)

        def _fill_one(i, carry):
            ones[pl.ds(i * 16, 16)] = one16
            return carry

        lax.fori_loop(0, CHUNK // 16, _fill_one, 0)

        plsc.subcore_barrier()

        base0 = wid * steps * CHUNK

        def _fetch(t, didx, sem):
            pltpu.async_copy(dst_hbm.at[pl.ds(base0 + t * CHUNK, CHUNK)],
                             didx, sem)

        def _wait(didx, sem):
            pltpu.make_async_copy(dst_hbm.at[pl.ds(base0, CHUNK)],
                                  didx, sem).wait()

        def _step(t, carry):
            pltpu.sync_copy(dst_hbm.at[pl.ds(base0 + t * CHUNK, CHUNK)], didx_a)
            pltpu.sync_copy(ones, acc.at[didx_a], add=True)
            return carry

        lax.fori_loop(0, steps, _step, 0)

        plsc.subcore_barrier()

        def _out(i, carry):
            off = s * stripe + i * CHUNK
            pltpu.sync_copy(acc.at[pl.ds(off, CHUNK)],
                            out_hbm.at[pl.ds(c * n_pad + off, CHUNK)])
            return carry

        lax.fori_loop(0, stripe // CHUNK, _out, 0)

    return deg_kernel


KF_PAD = 4
FAST_C = 1
FAST_FRAC = 0.6


def _core_split(chunks_per_pair, frac):
    cf = int(round(chunks_per_pair * frac / 2.0)) * 2
    cf = min(max(cf, 2), chunks_per_pair - 2)
    return cf, chunks_per_pair - cf


def _make_agg(e_pad, n_pad, d):
    chunks_pair = e_pad // (NS * CHUNK)
    cf, cs = _core_split(chunks_pair, FAST_FRAC if d > 16 else 0.5)
    stripe = n_pad // NS

    @functools.partial(
        pl.kernel,
        out_type=jax.ShapeDtypeStruct((NC * n_pad, d), jnp.float32),
        mesh=_sc_mesh(),
        scratch_types=[
            pltpu.VMEM((CHUNK,), jnp.int32),
            pltpu.VMEM((CHUNK,), jnp.int32),
            pltpu.VMEM((CHUNK, d), jnp.float32),
            pltpu.VMEM((CHUNK,), jnp.int32),
            pltpu.VMEM((CHUNK,), jnp.int32),
            pltpu.VMEM((CHUNK, d), jnp.float32),
            pltpu.VMEM_SHARED((n_pad, d), jnp.float32),
            pltpu.SemaphoreType.DMA,
            pltpu.SemaphoreType.DMA,
        ],
        compiler_params=pltpu.CompilerParams(use_tc_tiling_on_sc=False),
    )
    def agg_kernel(src_hbm, dst_hbm, g_hbm, out_hbm,
                   sa, da, ra, sb, db, rb, acc, sem_a, sem_b):
        c = lax.axis_index("c")
        s = lax.axis_index("s")

        zero16 = jnp.zeros((16,), jnp.float32)
        vecs_per_row = d // 16

        def _zrow(i, carry):
            r = i // vecs_per_row
            q = i % vecs_per_row
            ra[r, pl.ds(q * 16, 16)] = zero16
            return carry

        lax.fori_loop(0, CHUNK * vecs_per_row, _zrow, 0)

        def _zacc(i, carry):
            pltpu.sync_copy(ra, acc.at[pl.ds(s * stripe + i * CHUNK, CHUNK)])
            return carry

        lax.fori_loop(0, stripe // CHUNK, _zacc, 0)

        plsc.subcore_barrier()

        def _fetch(base0, t, sidx, didx, rows, sem):
            base = base0 + t * CHUNK
            pltpu.sync_copy(src_hbm.at[pl.ds(base, CHUNK)], sidx)
            pltpu.sync_copy(dst_hbm.at[pl.ds(base, CHUNK)], didx)
            pltpu.async_copy(g_hbm.at[sidx], rows, sem)

        def _wait(sidx, rows, sem):
            pltpu.make_async_copy(g_hbm.at[sidx], rows, sem).wait()

        def _serial_loop(base0, n):
            def _step(t, carry):
                _fetch(base0, t, sa, da, ra, sem_a)
                _wait(sa, ra, sem_a)
                pltpu.sync_copy(ra, acc.at[da], add=True)
                return carry

            lax.fori_loop(0, n, _step, 0)

        @pl.when(c == FAST_C)
        def _fast_path():
            _serial_loop(s * cf * CHUNK, cf)

        @pl.when(c != FAST_C)
        def _slow_path():
            _serial_loop((NS * cf + s * cs) * CHUNK, cs)

        plsc.subcore_barrier()

        def _out(i, carry):
            off = s * stripe + i * CHUNK
            pltpu.sync_copy(acc.at[pl.ds(off, CHUNK)],
                            out_hbm.at[pl.ds(c * n_pad + off, CHUNK)])
            return carry

        lax.fori_loop(0, stripe // CHUNK, _out, 0)

    return agg_kernel


def _tc1_body(n, degp, x, w1, dinv_o, g1_o):
    deg = degp[0] + degp[1] + 1.0
    dinv = lax.rsqrt(deg)
    dinv_o[...] = dinv
    g1_o[...] = jnp.dot(x[...], w1[...], preferred_element_type=jnp.float32) * dinv


def _tc2_body(n, a1, g1, dinv, b1, bn1g, bn1b, w2, wres, bres, g2_o, res_o):
    p = g1.shape[0]
    dv = dinv[...]
    s1 = dv * (a1[0] + a1[1] + g1[...]) + b1[...]
    rid = lax.broadcasted_iota(jnp.int32, (p, 1), 0)
    valid = rid < n
    s1m = jnp.where(valid, s1, 0.0)
    mean = jnp.sum(s1m, axis=0, keepdims=True) / n
    dlt = jnp.where(valid, s1 - mean, 0.0)
    var = jnp.sum(dlt * dlt, axis=0, keepdims=True) / n
    x1 = bn1g[...] * (s1 - mean) * lax.rsqrt(var + EPS) + bn1b[...]
    x1 = jnp.where(valid, jnp.maximum(x1, 0.0), 0.0)
    g2_o[...] = jnp.dot(x1, w2[...], preferred_element_type=jnp.float32) * dv
    res_o[...] = jnp.dot(x1, wres[...], preferred_element_type=jnp.float32) + bres[...]


def _tc3_body(n, a2, g2, res, dinv, b2, bn2g, bn2b, lng, lnb, w3p, g3_o):
    p = g2.shape[0]
    dv = dinv[...]
    s2 = dv * (a2[0] + a2[1] + g2[...]) + b2[...]
    rid = lax.broadcasted_iota(jnp.int32, (p, 1), 0)
    valid = rid < n
    s2m = jnp.where(valid, s2, 0.0)
    mean = jnp.sum(s2m, axis=0, keepdims=True) / n
    dlt = jnp.where(valid, s2 - mean, 0.0)
    var = jnp.sum(dlt * dlt, axis=0, keepdims=True) / n
    x2 = bn2g[...] * (s2 - mean) * lax.rsqrt(var + EPS) + bn2b[...]
    x2 = jnp.where(valid, jnp.maximum(x2, 0.0), 0.0)
    xr = res[...] + x2
    m = jnp.mean(xr, axis=1, keepdims=True)
    v = jnp.mean((xr - m) * (xr - m), axis=1, keepdims=True)
    xr = lng[...] * (xr - m) * lax.rsqrt(v + EPS) + lnb[...]
    xr = jnp.where(valid, jnp.maximum(xr, 0.0), 0.0)
    g3_o[...] = jnp.dot(xr, w3p[...], preferred_element_type=jnp.float32) * dv


def _tc4_body(a3, g3, dinv, b3p, out_o):
    p, w = g3.shape
    o = dinv[...] * (a3[0] + a3[1] + g3[...]) + b3p[...]
    cid = lax.broadcasted_iota(jnp.int32, (p, w), 1)
    cm = cid < 2
    om = jnp.where(cm, o, -jnp.inf)
    mx = jnp.max(om, axis=1, keepdims=True)
    e = jnp.where(cm, jnp.exp(o - mx), 0.0)
    lse = mx + jnp.log(jnp.sum(e, axis=1, keepdims=True))
    out_o[...] = o - lse


def _f32(*shapes):
    return [jax.ShapeDtypeStruct(sh, jnp.float32) for sh in shapes]


def kernel(x, edge_index, W1, b1, bn1_g, bn1_b, W2, b2, bn2_g, bn2_b,
           Wres, bres, ln_g, ln_b, W3, b3):
    n, d0 = x.shape
    e = edge_index.shape[1]
    d1 = W2.shape[1]
    d3 = 16
    p = -(-n // (16 * NS)) * (16 * NS)
    e_pad = -(-e // (NW * CHUNK * KF_PAD)) * (NW * CHUNK * KF_PAD)

    src = jnp.concatenate(
        [edge_index[0], jnp.zeros((e_pad - e,), jnp.int32)])
    dst = jnp.concatenate(
        [edge_index[1], jnp.full((e_pad - e,), n, jnp.int32)])
    xp = jnp.pad(x, ((0, p - n), (0, 0)))
    w3p = jnp.pad(W3, ((0, 0), (0, d3 - W3.shape[1])))
    b3p = jnp.pad(b3, (0, d3 - b3.shape[0]))

    degp = _make_deg(e_pad, p)(dst).reshape(NC, p, 1)

    dinv, g1 = pl.pallas_call(
        functools.partial(_tc1_body, n),
        out_shape=_f32((p, 1), (p, d0)),
    )(degp, xp, W1)

    a1 = _make_agg(e_pad, p, d0)(src, dst, g1).reshape(NC, p, d0)

    g2, res = pl.pallas_call(
        functools.partial(_tc2_body, n),
        out_shape=_f32((p, d1), (p, d1)),
    )(a1, g1, dinv, b1, bn1_g, bn1_b, W2, Wres, bres)

    a2 = _make_agg(e_pad, p, d1)(src, dst, g2).reshape(NC, p, d1)

    g3 = pl.pallas_call(
        functools.partial(_tc3_body, n),
        out_shape=_f32((p, d3))[0],
    )(a2, g2, res, dinv, b2, bn2_g, bn2_b, ln_g, ln_b, w3p)

    a3 = _make_agg(e_pad, p, d3)(src, dst, g3).reshape(NC, p, d3)

    outp = pl.pallas_call(
        _tc4_body,
        out_shape=_f32((p, d3))[0],
    )(a3, g3, dinv, b3p)

    return outp[:n, :2]


# speedup vs baseline: 1.7104x; 1.7104x over previous
import functools

import jax
import jax.numpy as jnp
from jax import lax
from jax.experimental import pallas as pl
from jax.experimental.pallas import tpu as pltpu
from jax.experimental.pallas import tpu_sc as plsc

NC = 2
NS = 16
NW = NC * NS
CHUNK = 128
EPS = 1e-5


def _sc_mesh():
    return plsc.VectorSubcoreMesh(core_axis_name="c", subcore_axis_name="s",
                                  num_cores=NC, num_subcores=NS)


def _make_deg(e_pad, n_pad):
    k_pw = e_pad // (NW * CHUNK)
    stripe = n_pad // NS

    @functools.partial(
        pl.kernel,
        out_type=jax.ShapeDtypeStruct((NC * n_pad,), jnp.float32),
        mesh=_sc_mesh(),
        scratch_types=[
            pltpu.VMEM((k_pw, CHUNK), jnp.int32),
            pltpu.VMEM((CHUNK,), jnp.float32),
            pltpu.VMEM_SHARED((n_pad,), jnp.float32),
            pltpu.SemaphoreType.DMA,
        ],
    )
    def deg_kernel(dst_hbm, out_hbm, didx, ones, acc, sem):
        c = lax.axis_index("c")
        s = lax.axis_index("s")
        wid = s * NC + c

        pltpu.sync_copy(dst_hbm.at[pl.ds(wid * k_pw, k_pw)], didx)

        one16 = jnp.ones((16,), jnp.float32)
        zero16 = jnp.zeros((16,), jnp.float32)

        def _fill_zero(i, carry):
            ones[pl.ds(i * 16, 16)] = zero16
            return carry

        lax.fori_loop(0, CHUNK // 16, _fill_zero, 0)

        def _zacc(i, carry):
            pltpu.sync_copy(ones, acc.at[pl.ds(s * stripe + i * CHUNK, CHUNK)])
            return carry

        lax.fori_loop(0, stripe // CHUNK, _zacc, 0)

        def _fill_one(i, carry):
            ones[pl.ds(i * 16, 16)] = one16
            return carry

        lax.fori_loop(0, CHUNK // 16, _fill_one, 0)

        plsc.subcore_barrier()

        def _step(t, carry):
            pltpu.sync_copy(ones, acc.at[didx.at[t]], add=True)
            return carry

        lax.fori_loop(0, k_pw, _step, 0)

        plsc.subcore_barrier()

        def _out(i, carry):
            off = s * stripe + i * CHUNK
            pltpu.sync_copy(acc.at[pl.ds(off, CHUNK)],
                            out_hbm.at[pl.ds(c * n_pad + off, CHUNK)])
            return carry

        lax.fori_loop(0, stripe // CHUNK, _out, 0)

    return deg_kernel


KF_PAD = 4


def _make_agg(e_pad, n_pad, d):
    k_pw = e_pad // (NW * CHUNK)
    stripe = n_pad // NS
    stage_g = d <= 64

    scratch = [
        pltpu.VMEM((k_pw, CHUNK), jnp.int32),
        pltpu.VMEM((k_pw, CHUNK), jnp.int32),
        pltpu.VMEM((CHUNK, d), jnp.float32),
        pltpu.VMEM_SHARED((n_pad, d), jnp.float32),
        pltpu.SemaphoreType.DMA,
    ]
    if stage_g:
        scratch.append(pltpu.VMEM_SHARED((n_pad, d), jnp.float32))

    @functools.partial(
        pl.kernel,
        out_type=jax.ShapeDtypeStruct((NC * n_pad, d), jnp.float32),
        mesh=_sc_mesh(),
        scratch_types=scratch,
        compiler_params=pltpu.CompilerParams(use_tc_tiling_on_sc=False),
    )
    def agg_kernel(src_hbm, dst_hbm, g_hbm, out_hbm, *refs):
        if stage_g:
            sidx, didx, rows, acc, sem, gtab = refs
        else:
            sidx, didx, rows, acc, sem = refs
            gtab = g_hbm
        c = lax.axis_index("c")
        s = lax.axis_index("s")
        wid = s * NC + c

        pltpu.sync_copy(src_hbm.at[pl.ds(wid * k_pw, k_pw)], sidx)
        pltpu.sync_copy(dst_hbm.at[pl.ds(wid * k_pw, k_pw)], didx)

        zero16 = jnp.zeros((16,), jnp.float32)
        vecs_per_row = d // 16

        def _zrow(i, carry):
            r = i // vecs_per_row
            q = i % vecs_per_row
            rows[r, pl.ds(q * 16, 16)] = zero16
            return carry

        lax.fori_loop(0, CHUNK * vecs_per_row, _zrow, 0)

        def _zacc(i, carry):
            pltpu.sync_copy(rows, acc.at[pl.ds(s * stripe + i * CHUNK, CHUNK)])
            return carry

        lax.fori_loop(0, stripe // CHUNK, _zacc, 0)

        if stage_g:
            pltpu.sync_copy(g_hbm.at[pl.ds(s * stripe, stripe)],
                            gtab.at[pl.ds(s * stripe, stripe)])

        plsc.subcore_barrier()

        def _step(t, carry):
            pltpu.async_copy(gtab.at[sidx.at[t]], rows, sem).wait()
            pltpu.sync_copy(rows, acc.at[didx.at[t]], add=True)
            return carry

        lax.fori_loop(0, k_pw, _step, 0)

        plsc.subcore_barrier()

        def _out(i, carry):
            off = s * stripe + i * CHUNK
            pltpu.sync_copy(acc.at[pl.ds(off, CHUNK)],
                            out_hbm.at[pl.ds(c * n_pad + off, CHUNK)])
            return carry

        lax.fori_loop(0, stripe // CHUNK, _out, 0)

    return agg_kernel


def _tc1_body(n, degp, x, w1, dinv_o, g1_o):
    deg = degp[0] + degp[1] + 1.0
    dinv = lax.rsqrt(deg)
    dinv_o[...] = dinv
    g1_o[...] = jnp.dot(x[...], w1[...], preferred_element_type=jnp.float32) * dinv


def _tc2_body(n, a1, g1, dinv, b1, bn1g, bn1b, w2, wres, bres, g2_o, res_o):
    p = g1.shape[0]
    dv = dinv[...]
    s1 = dv * (a1[0] + a1[1] + g1[...]) + b1[...]
    rid = lax.broadcasted_iota(jnp.int32, (p, 1), 0)
    valid = rid < n
    s1m = jnp.where(valid, s1, 0.0)
    mean = jnp.sum(s1m, axis=0, keepdims=True) / n
    dlt = jnp.where(valid, s1 - mean, 0.0)
    var = jnp.sum(dlt * dlt, axis=0, keepdims=True) / n
    x1 = bn1g[...] * (s1 - mean) * lax.rsqrt(var + EPS) + bn1b[...]
    x1 = jnp.where(valid, jnp.maximum(x1, 0.0), 0.0)
    g2_o[...] = jnp.dot(x1, w2[...], preferred_element_type=jnp.float32) * dv
    res_o[...] = jnp.dot(x1, wres[...], preferred_element_type=jnp.float32) + bres[...]


def _tc3_body(n, a2, g2, res, dinv, b2, bn2g, bn2b, lng, lnb, w3p, g3_o):
    p = g2.shape[0]
    dv = dinv[...]
    s2 = dv * (a2[0] + a2[1] + g2[...]) + b2[...]
    rid = lax.broadcasted_iota(jnp.int32, (p, 1), 0)
    valid = rid < n
    s2m = jnp.where(valid, s2, 0.0)
    mean = jnp.sum(s2m, axis=0, keepdims=True) / n
    dlt = jnp.where(valid, s2 - mean, 0.0)
    var = jnp.sum(dlt * dlt, axis=0, keepdims=True) / n
    x2 = bn2g[...] * (s2 - mean) * lax.rsqrt(var + EPS) + bn2b[...]
    x2 = jnp.where(valid, jnp.maximum(x2, 0.0), 0.0)
    xr = res[...] + x2
    m = jnp.mean(xr, axis=1, keepdims=True)
    v = jnp.mean((xr - m) * (xr - m), axis=1, keepdims=True)
    xr = lng[...] * (xr - m) * lax.rsqrt(v + EPS) + lnb[...]
    xr = jnp.where(valid, jnp.maximum(xr, 0.0), 0.0)
    g3_o[...] = jnp.dot(xr, w3p[...], preferred_element_type=jnp.float32) * dv


def _tc4_body(a3, g3, dinv, b3p, out_o):
    p, w = g3.shape
    o = dinv[...] * (a3[0] + a3[1] + g3[...]) + b3p[...]
    cid = lax.broadcasted_iota(jnp.int32, (p, w), 1)
    cm = cid < 2
    om = jnp.where(cm, o, -jnp.inf)
    mx = jnp.max(om, axis=1, keepdims=True)
    e = jnp.where(cm, jnp.exp(o - mx), 0.0)
    lse = mx + jnp.log(jnp.sum(e, axis=1, keepdims=True))
    out_o[...] = o - lse


def _f32(*shapes):
    return [jax.ShapeDtypeStruct(sh, jnp.float32) for sh in shapes]


def kernel(x, edge_index, W1, b1, bn1_g, bn1_b, W2, b2, bn2_g, bn2_b,
           Wres, bres, ln_g, ln_b, W3, b3):
    n, d0 = x.shape
    e = edge_index.shape[1]
    d1 = W2.shape[1]
    d3 = 16
    p = -(-n // (16 * NS)) * (16 * NS)
    e_pad = -(-e // (NW * CHUNK * KF_PAD)) * (NW * CHUNK * KF_PAD)

    src = jnp.concatenate(
        [edge_index[0], jnp.zeros((e_pad - e,), jnp.int32)]
    ).reshape(e_pad // CHUNK, CHUNK)
    dst = jnp.concatenate(
        [edge_index[1], jnp.full((e_pad - e,), n, jnp.int32)]
    ).reshape(e_pad // CHUNK, CHUNK)
    xp = jnp.pad(x, ((0, p - n), (0, 0)))
    w3p = jnp.pad(W3, ((0, 0), (0, d3 - W3.shape[1])))
    b3p = jnp.pad(b3, (0, d3 - b3.shape[0]))

    degp = _make_deg(e_pad, p)(dst).reshape(NC, p, 1)

    dinv, g1 = pl.pallas_call(
        functools.partial(_tc1_body, n),
        out_shape=_f32((p, 1), (p, d0)),
    )(degp, xp, W1)

    a1 = _make_agg(e_pad, p, d0)(src, dst, g1).reshape(NC, p, d0)

    g2, res = pl.pallas_call(
        functools.partial(_tc2_body, n),
        out_shape=_f32((p, d1), (p, d1)),
    )(a1, g1, dinv, b1, bn1_g, bn1_b, W2, Wres, bres)

    a2 = _make_agg(e_pad, p, d1)(src, dst, g2).reshape(NC, p, d1)

    g3 = pl.pallas_call(
        functools.partial(_tc3_body, n),
        out_shape=_f32((p, d3))[0],
    )(a2, g2, res, dinv, b2, bn2_g, bn2_b, ln_g, ln_b, w3p)

    a3 = _make_agg(e_pad, p, d3)(src, dst, g3).reshape(NC, p, d3)

    outp = pl.pallas_call(
        _tc4_body,
        out_shape=_f32((p, d3))[0],
    )(a3, g3, dinv, b3p)

    return outp[:n, :2]


# speedup vs baseline: 2.3405x; 1.3684x over previous
import functools

import jax
import jax.numpy as jnp
from jax import lax
from jax.experimental import pallas as pl
from jax.experimental.pallas import tpu as pltpu
from jax.experimental.pallas import tpu_sc as plsc

NC = 2
NS = 16
NW = NC * NS
CHUNK = 128
EPS = 1e-5


def _sc_mesh():
    return plsc.VectorSubcoreMesh(core_axis_name="c", subcore_axis_name="s",
                                  num_cores=NC, num_subcores=NS)


def _make_deg(e_pad, n_pad):
    k_pw = e_pad // (NW * CHUNK)
    stripe = n_pad // NS

    @functools.partial(
        pl.kernel,
        out_type=jax.ShapeDtypeStruct((NC * n_pad,), jnp.float32),
        mesh=_sc_mesh(),
        scratch_types=[
            pltpu.VMEM((k_pw, CHUNK), jnp.int32),
            pltpu.VMEM((CHUNK,), jnp.float32),
            pltpu.VMEM_SHARED((n_pad,), jnp.float32),
            pltpu.SemaphoreType.DMA,
        ],
    )
    def deg_kernel(dst_hbm, out_hbm, didx, ones, acc, sem):
        c = lax.axis_index("c")
        s = lax.axis_index("s")
        wid = s * NC + c

        pltpu.sync_copy(dst_hbm.at[pl.ds(wid * k_pw, k_pw)], didx)

        one16 = jnp.ones((16,), jnp.float32)
        zero16 = jnp.zeros((16,), jnp.float32)

        def _fill_zero(i, carry):
            ones[pl.ds(i * 16, 16)] = zero16
            return carry

        lax.fori_loop(0, CHUNK // 16, _fill_zero, 0)

        def _zacc(i, carry):
            pltpu.sync_copy(ones, acc.at[pl.ds(s * stripe + i * CHUNK, CHUNK)])
            return carry

        lax.fori_loop(0, stripe // CHUNK, _zacc, 0)

        def _fill_one(i, carry):
            ones[pl.ds(i * 16, 16)] = one16
            return carry

        lax.fori_loop(0, CHUNK // 16, _fill_one, 0)

        plsc.subcore_barrier()

        def _step(t, carry):
            pltpu.sync_copy(ones, acc.at[didx.at[t]], add=True)
            return carry

        lax.fori_loop(0, k_pw, _step, 0)

        plsc.subcore_barrier()

        def _out(i, carry):
            off = s * stripe + i * CHUNK
            pltpu.sync_copy(acc.at[pl.ds(off, CHUNK)],
                            out_hbm.at[pl.ds(c * n_pad + off, CHUNK)])
            return carry

        lax.fori_loop(0, stripe // CHUNK, _out, 0)

    return deg_kernel


KF_PAD = 4


def _make_agg(e_pad, n_pad, d):
    k_pw = e_pad // (NW * CHUNK)
    stripe = n_pad // NS
    stage_g = d <= 64

    scratch = [
        pltpu.VMEM((k_pw, CHUNK), jnp.int32),
        pltpu.VMEM((k_pw, CHUNK), jnp.int32),
        pltpu.VMEM((CHUNK, d), jnp.float32),
        pltpu.VMEM_SHARED((n_pad, d), jnp.float32),
        pltpu.SemaphoreType.DMA,
    ]
    if stage_g:
        scratch.append(pltpu.VMEM_SHARED((n_pad, d), jnp.float32))

    @functools.partial(
        pl.kernel,
        out_type=jax.ShapeDtypeStruct((NC * n_pad, d), jnp.float32),
        mesh=_sc_mesh(),
        scratch_types=scratch,
        compiler_params=pltpu.CompilerParams(use_tc_tiling_on_sc=False),
    )
    def agg_kernel(src_hbm, dst_hbm, g_hbm, out_hbm, *refs):
        if stage_g:
            sidx, didx, rows, acc, sem, gtab = refs
        else:
            sidx, didx, rows, acc, sem = refs
            gtab = g_hbm
        c = lax.axis_index("c")
        s = lax.axis_index("s")
        wid = s * NC + c

        pltpu.sync_copy(src_hbm.at[pl.ds(wid * k_pw, k_pw)], sidx)
        pltpu.sync_copy(dst_hbm.at[pl.ds(wid * k_pw, k_pw)], didx)

        zero16 = jnp.zeros((16,), jnp.float32)
        vecs_per_row = d // 16

        def _zrow(i, carry):
            r = i // vecs_per_row
            q = i % vecs_per_row
            rows[r, pl.ds(q * 16, 16)] = zero16
            return carry

        lax.fori_loop(0, CHUNK * vecs_per_row, _zrow, 0)

        def _zacc(i, carry):
            pltpu.sync_copy(rows, acc.at[pl.ds(s * stripe + i * CHUNK, CHUNK)])
            return carry

        lax.fori_loop(0, stripe // CHUNK, _zacc, 0)

        if stage_g:
            pltpu.sync_copy(g_hbm.at[pl.ds(s * stripe, stripe)],
                            gtab.at[pl.ds(s * stripe, stripe)])

        plsc.subcore_barrier()

        def _step(t, carry):
            pltpu.async_copy(gtab.at[sidx.at[t]], rows, sem).wait()
            pltpu.sync_copy(rows, acc.at[didx.at[t]], add=True)
            return carry

        lax.fori_loop(0, k_pw, _step, 0)

        plsc.subcore_barrier()

        def _out(i, carry):
            off = s * stripe + i * CHUNK
            pltpu.sync_copy(acc.at[pl.ds(off, CHUNK)],
                            out_hbm.at[pl.ds(c * n_pad + off, CHUNK)])
            return carry

        lax.fori_loop(0, stripe // CHUNK, _out, 0)

    return agg_kernel


def _tc1_body(n, degp, x, w1, dinv_o, g1_o):
    deg = degp[0] + degp[1] + 1.0
    dinv = lax.rsqrt(deg)
    dinv_o[...] = dinv
    g1_o[...] = jnp.dot(x[...], w1[...], preferred_element_type=jnp.float32) * dinv


def _tc2_body(n, a1, g1, dinv, b1, bn1g, bn1b, w2, wres, bres, g2_o, res_o):
    p = g1.shape[0]
    dv = dinv[...]
    s1 = dv * (a1[0] + a1[1] + g1[...]) + b1[...]
    rid = lax.broadcasted_iota(jnp.int32, (p, 1), 0)
    valid = rid < n
    s1m = jnp.where(valid, s1, 0.0)
    mean = jnp.sum(s1m, axis=0, keepdims=True) / n
    dlt = jnp.where(valid, s1 - mean, 0.0)
    var = jnp.sum(dlt * dlt, axis=0, keepdims=True) / n
    x1 = bn1g[...] * (s1 - mean) * lax.rsqrt(var + EPS) + bn1b[...]
    x1 = jnp.where(valid, jnp.maximum(x1, 0.0), 0.0)
    g2_o[...] = jnp.dot(x1, w2[...], preferred_element_type=jnp.float32) * dv
    res_o[...] = jnp.dot(x1, wres[...], preferred_element_type=jnp.float32) + bres[...]


def _tc3_body(n, a2, g2, res, dinv, b2, bn2g, bn2b, lng, lnb, w3p, g3_o):
    p = g2.shape[0]
    dv = dinv[...]
    s2 = dv * (a2[0] + a2[1] + g2[...]) + b2[...]
    rid = lax.broadcasted_iota(jnp.int32, (p, 1), 0)
    valid = rid < n
    s2m = jnp.where(valid, s2, 0.0)
    mean = jnp.sum(s2m, axis=0, keepdims=True) / n
    dlt = jnp.where(valid, s2 - mean, 0.0)
    var = jnp.sum(dlt * dlt, axis=0, keepdims=True) / n
    x2 = bn2g[...] * (s2 - mean) * lax.rsqrt(var + EPS) + bn2b[...]
    x2 = jnp.where(valid, jnp.maximum(x2, 0.0), 0.0)
    xr = res[...] + x2
    m = jnp.mean(xr, axis=1, keepdims=True)
    v = jnp.mean((xr - m) * (xr - m), axis=1, keepdims=True)
    xr = lng[...] * (xr - m) * lax.rsqrt(v + EPS) + lnb[...]
    xr = jnp.where(valid, jnp.maximum(xr, 0.0), 0.0)
    g3_o[...] = jnp.dot(xr, w3p[...], preferred_element_type=jnp.float32) * dv


def _tc4_body(a3, g3, dinv, b3p, out_o):
    p, w = g3.shape
    o = dinv[...] * (a3[0] + a3[1] + g3[...]) + b3p[...]
    cid = lax.broadcasted_iota(jnp.int32, (p, w), 1)
    cm = cid < 2
    om = jnp.where(cm, o, -jnp.inf)
    mx = jnp.max(om, axis=1, keepdims=True)
    e = jnp.where(cm, jnp.exp(o - mx), 0.0)
    lse = mx + jnp.log(jnp.sum(e, axis=1, keepdims=True))
    out_o[...] = o - lse


def _f32(*shapes):
    return [jax.ShapeDtypeStruct(sh, jnp.float32) for sh in shapes]


def kernel(x, edge_index, W1, b1, bn1_g, bn1_b, W2, b2, bn2_g, bn2_b,
           Wres, bres, ln_g, ln_b, W3, b3):
    n, d0 = x.shape
    e = edge_index.shape[1]
    d1 = W2.shape[1]
    d3 = 16
    p = -(-n // (16 * NS)) * (16 * NS)
    e_pad = -(-e // (NW * CHUNK * KF_PAD)) * (NW * CHUNK * KF_PAD)

    src = jnp.concatenate(
        [edge_index[0], jnp.zeros((e_pad - e,), jnp.int32)]
    ).reshape(e_pad // CHUNK, CHUNK)
    dst = jnp.concatenate(
        [edge_index[1], jnp.full((e_pad - e,), n, jnp.int32)]
    ).reshape(e_pad // CHUNK, CHUNK)
    xp = jnp.pad(x, ((0, p - n), (0, 0)))
    w3p = jnp.pad(W3, ((0, 0), (0, d3 - W3.shape[1])))
    b3p = jnp.pad(b3, (0, d3 - b3.shape[0]))

    degp = _make_deg(e_pad, p)(dst).reshape(NC, p, 1)

    dinv, g1 = pl.pallas_call(
        functools.partial(_tc1_body, n),
        out_shape=_f32((p, 1), (p, d0)),
    )(degp, xp, W1)

    dh = d0 // 2
    a1a = _make_agg(e_pad, p, dh)(src, dst, g1[:, :dh]).reshape(NC, p, dh)
    a1b = _make_agg(e_pad, p, dh)(src, dst, g1[:, dh:]).reshape(NC, p, dh)
    a1 = jnp.concatenate([a1a, a1b], axis=2)

    g2, res = pl.pallas_call(
        functools.partial(_tc2_body, n),
        out_shape=_f32((p, d1), (p, d1)),
    )(a1, g1, dinv, b1, bn1_g, bn1_b, W2, Wres, bres)

    a2 = _make_agg(e_pad, p, d1)(src, dst, g2).reshape(NC, p, d1)

    g3 = pl.pallas_call(
        functools.partial(_tc3_body, n),
        out_shape=_f32((p, d3))[0],
    )(a2, g2, res, dinv, b2, bn2_g, bn2_b, ln_g, ln_b, w3p)

    a3 = _make_agg(e_pad, p, d3)(src, dst, g3).reshape(NC, p, d3)

    outp = pl.pallas_call(
        _tc4_body,
        out_shape=_f32((p, d3))[0],
    )(a3, g3, dinv, b3p)

    return outp[:n, :2]


# speedup vs baseline: 2.8845x; 1.2324x over previous
import functools

import jax
import jax.numpy as jnp
from jax import lax
from jax.experimental import pallas as pl
from jax.experimental.pallas import tpu as pltpu
from jax.experimental.pallas import tpu_sc as plsc

NC = 2
NS = 16
NW = NC * NS
CHUNK = 128
EPS = 1e-5


def _sc_mesh():
    return plsc.VectorSubcoreMesh(core_axis_name="c", subcore_axis_name="s",
                                  num_cores=NC, num_subcores=NS)


def _make_deg(e_pad, n_pad):
    k_pw = e_pad // (NW * CHUNK)
    stripe = n_pad // NS

    @functools.partial(
        pl.kernel,
        out_type=jax.ShapeDtypeStruct((NC * n_pad,), jnp.float32),
        mesh=_sc_mesh(),
        scratch_types=[
            pltpu.VMEM((k_pw, CHUNK), jnp.int32),
            pltpu.VMEM((CHUNK,), jnp.float32),
            pltpu.VMEM_SHARED((n_pad,), jnp.float32),
            pltpu.SemaphoreType.DMA,
        ],
    )
    def deg_kernel(dst_hbm, out_hbm, didx, ones, acc, sem):
        c = lax.axis_index("c")
        s = lax.axis_index("s")
        wid = s * NC + c

        pltpu.sync_copy(dst_hbm.at[pl.ds(wid * k_pw, k_pw)], didx)

        one16 = jnp.ones((16,), jnp.float32)
        zero16 = jnp.zeros((16,), jnp.float32)

        def _fill_zero(i, carry):
            ones[pl.ds(i * 16, 16)] = zero16
            return carry

        lax.fori_loop(0, CHUNK // 16, _fill_zero, 0)

        def _zacc(i, carry):
            pltpu.sync_copy(ones, acc.at[pl.ds(s * stripe + i * CHUNK, CHUNK)])
            return carry

        lax.fori_loop(0, stripe // CHUNK, _zacc, 0)

        def _fill_one(i, carry):
            ones[pl.ds(i * 16, 16)] = one16
            return carry

        lax.fori_loop(0, CHUNK // 16, _fill_one, 0)

        plsc.subcore_barrier()

        def _step(t, carry):
            pltpu.sync_copy(ones, acc.at[didx.at[t]], add=True)
            return carry

        lax.fori_loop(0, k_pw, _step, 0)

        plsc.subcore_barrier()

        def _out(i, carry):
            off = s * stripe + i * CHUNK
            pltpu.sync_copy(acc.at[pl.ds(off, CHUNK)],
                            out_hbm.at[pl.ds(c * n_pad + off, CHUNK)])
            return carry

        lax.fori_loop(0, stripe // CHUNK, _out, 0)

    return deg_kernel


KF_PAD = 4


def _make_agg(e_pad, n_pad, d):
    k_pw = e_pad // (NW * CHUNK)
    stripe = n_pad // NS
    stage_g = d <= 64

    scratch = [
        pltpu.VMEM((k_pw, CHUNK), jnp.int32),
        pltpu.VMEM((k_pw, CHUNK), jnp.int32),
        pltpu.VMEM((CHUNK, d), jnp.float32),
        pltpu.VMEM((CHUNK, d), jnp.float32),
        pltpu.VMEM_SHARED((n_pad, d), jnp.float32),
        pltpu.SemaphoreType.DMA,
        pltpu.SemaphoreType.DMA,
    ]
    if stage_g:
        scratch.append(pltpu.VMEM_SHARED((n_pad, d), jnp.float32))

    @functools.partial(
        pl.kernel,
        out_type=jax.ShapeDtypeStruct((NC * n_pad, d), jnp.float32),
        mesh=_sc_mesh(),
        scratch_types=scratch,
        compiler_params=pltpu.CompilerParams(use_tc_tiling_on_sc=False),
    )
    def agg_kernel(src_hbm, dst_hbm, g_hbm, out_hbm, *refs):
        if stage_g:
            sidx, didx, ra, rb, acc, sem_a, sem_b, gtab = refs
        else:
            sidx, didx, ra, rb, acc, sem_a, sem_b = refs
            gtab = g_hbm
        rows = ra
        c = lax.axis_index("c")
        s = lax.axis_index("s")
        wid = s * NC + c

        pltpu.sync_copy(src_hbm.at[pl.ds(wid * k_pw, k_pw)], sidx)
        pltpu.sync_copy(dst_hbm.at[pl.ds(wid * k_pw, k_pw)], didx)

        zero16 = jnp.zeros((16,), jnp.float32)
        vecs_per_row = d // 16

        def _zrow(i, carry):
            r = i // vecs_per_row
            q = i % vecs_per_row
            rows[r, pl.ds(q * 16, 16)] = zero16
            return carry

        lax.fori_loop(0, CHUNK * vecs_per_row, _zrow, 0)

        def _zacc(i, carry):
            pltpu.sync_copy(rows, acc.at[pl.ds(s * stripe + i * CHUNK, CHUNK)])
            return carry

        lax.fori_loop(0, stripe // CHUNK, _zacc, 0)

        if stage_g:
            pltpu.sync_copy(g_hbm.at[pl.ds(s * stripe, stripe)],
                            gtab.at[pl.ds(s * stripe, stripe)])

        plsc.subcore_barrier()

        def _gather(t, buf, sem):
            pltpu.async_copy(gtab.at[sidx.at[t]], buf, sem)

        def _gwait(t, buf, sem):
            pltpu.make_async_copy(gtab.at[sidx.at[t]], buf, sem).wait()

        _gather(0, ra, sem_a)
        _gather(1, rb, sem_b)

        def _step(i, carry):
            _gwait(2 * i, ra, sem_a)
            pltpu.sync_copy(ra, acc.at[didx.at[2 * i]], add=True)
            _gather(2 * i + 2, ra, sem_a)
            _gwait(2 * i + 1, rb, sem_b)
            pltpu.sync_copy(rb, acc.at[didx.at[2 * i + 1]], add=True)
            _gather(2 * i + 3, rb, sem_b)
            return carry

        lax.fori_loop(0, k_pw // 2 - 1, _step, 0)
        _gwait(k_pw - 2, ra, sem_a)
        pltpu.sync_copy(ra, acc.at[didx.at[k_pw - 2]], add=True)
        _gwait(k_pw - 1, rb, sem_b)
        pltpu.sync_copy(rb, acc.at[didx.at[k_pw - 1]], add=True)

        plsc.subcore_barrier()

        def _out(i, carry):
            off = s * stripe + i * CHUNK
            pltpu.sync_copy(acc.at[pl.ds(off, CHUNK)],
                            out_hbm.at[pl.ds(c * n_pad + off, CHUNK)])
            return carry

        lax.fori_loop(0, stripe // CHUNK, _out, 0)

    return agg_kernel


def _tc1_body(n, degp, x, w1, dinv_o, g1_o):
    deg = degp[0] + degp[1] + 1.0
    dinv = lax.rsqrt(deg)
    dinv_o[...] = dinv
    g1_o[...] = jnp.dot(x[...], w1[...], preferred_element_type=jnp.float32) * dinv


def _tc2_body(n, a1, g1, dinv, b1, bn1g, bn1b, w2, wres, bres, g2_o, res_o):
    p = g1.shape[0]
    dv = dinv[...]
    s1 = dv * (a1[0] + a1[1] + g1[...]) + b1[...]
    rid = lax.broadcasted_iota(jnp.int32, (p, 1), 0)
    valid = rid < n
    s1m = jnp.where(valid, s1, 0.0)
    mean = jnp.sum(s1m, axis=0, keepdims=True) / n
    dlt = jnp.where(valid, s1 - mean, 0.0)
    var = jnp.sum(dlt * dlt, axis=0, keepdims=True) / n
    x1 = bn1g[...] * (s1 - mean) * lax.rsqrt(var + EPS) + bn1b[...]
    x1 = jnp.where(valid, jnp.maximum(x1, 0.0), 0.0)
    g2_o[...] = jnp.dot(x1, w2[...], preferred_element_type=jnp.float32) * dv
    res_o[...] = jnp.dot(x1, wres[...], preferred_element_type=jnp.float32) + bres[...]


def _tc3_body(n, a2, g2, res, dinv, b2, bn2g, bn2b, lng, lnb, w3p, g3_o):
    p = g2.shape[0]
    dv = dinv[...]
    s2 = dv * (a2[0] + a2[1] + g2[...]) + b2[...]
    rid = lax.broadcasted_iota(jnp.int32, (p, 1), 0)
    valid = rid < n
    s2m = jnp.where(valid, s2, 0.0)
    mean = jnp.sum(s2m, axis=0, keepdims=True) / n
    dlt = jnp.where(valid, s2 - mean, 0.0)
    var = jnp.sum(dlt * dlt, axis=0, keepdims=True) / n
    x2 = bn2g[...] * (s2 - mean) * lax.rsqrt(var + EPS) + bn2b[...]
    x2 = jnp.where(valid, jnp.maximum(x2, 0.0), 0.0)
    xr = res[...] + x2
    m = jnp.mean(xr, axis=1, keepdims=True)
    v = jnp.mean((xr - m) * (xr - m), axis=1, keepdims=True)
    xr = lng[...] * (xr - m) * lax.rsqrt(v + EPS) + lnb[...]
    xr = jnp.where(valid, jnp.maximum(xr, 0.0), 0.0)
    g3_o[...] = jnp.dot(xr, w3p[...], preferred_element_type=jnp.float32) * dv


def _tc4_body(a3, g3, dinv, b3p, out_o):
    p, w = g3.shape
    o = dinv[...] * (a3[0] + a3[1] + g3[...]) + b3p[...]
    cid = lax.broadcasted_iota(jnp.int32, (p, w), 1)
    cm = cid < 2
    om = jnp.where(cm, o, -jnp.inf)
    mx = jnp.max(om, axis=1, keepdims=True)
    e = jnp.where(cm, jnp.exp(o - mx), 0.0)
    lse = mx + jnp.log(jnp.sum(e, axis=1, keepdims=True))
    out_o[...] = o - lse


def _f32(*shapes):
    return [jax.ShapeDtypeStruct(sh, jnp.float32) for sh in shapes]


def kernel(x, edge_index, W1, b1, bn1_g, bn1_b, W2, b2, bn2_g, bn2_b,
           Wres, bres, ln_g, ln_b, W3, b3):
    n, d0 = x.shape
    e = edge_index.shape[1]
    d1 = W2.shape[1]
    d3 = 16
    p = -(-n // (16 * NS)) * (16 * NS)
    e_pad = -(-e // (NW * CHUNK * KF_PAD)) * (NW * CHUNK * KF_PAD)

    src = jnp.concatenate(
        [edge_index[0], jnp.zeros((e_pad - e,), jnp.int32)]
    ).reshape(e_pad // CHUNK, CHUNK)
    dst = jnp.concatenate(
        [edge_index[1], jnp.full((e_pad - e,), n, jnp.int32)]
    ).reshape(e_pad // CHUNK, CHUNK)
    xp = jnp.pad(x, ((0, p - n), (0, 0)))
    w3p = jnp.pad(W3, ((0, 0), (0, d3 - W3.shape[1])))
    b3p = jnp.pad(b3, (0, d3 - b3.shape[0]))

    degp = _make_deg(e_pad, p)(dst).reshape(NC, p, 1)

    dinv, g1 = pl.pallas_call(
        functools.partial(_tc1_body, n),
        out_shape=_f32((p, 1), (p, d0)),
    )(degp, xp, W1)

    dh = d0 // 2
    a1a = _make_agg(e_pad, p, dh)(src, dst, g1[:, :dh]).reshape(NC, p, dh)
    a1b = _make_agg(e_pad, p, dh)(src, dst, g1[:, dh:]).reshape(NC, p, dh)
    a1 = jnp.concatenate([a1a, a1b], axis=2)

    g2, res = pl.pallas_call(
        functools.partial(_tc2_body, n),
        out_shape=_f32((p, d1), (p, d1)),
    )(a1, g1, dinv, b1, bn1_g, bn1_b, W2, Wres, bres)

    a2 = _make_agg(e_pad, p, d1)(src, dst, g2).reshape(NC, p, d1)

    g3 = pl.pallas_call(
        functools.partial(_tc3_body, n),
        out_shape=_f32((p, d3))[0],
    )(a2, g2, res, dinv, b2, bn2_g, bn2_b, ln_g, ln_b, w3p)

    a3 = _make_agg(e_pad, p, d3)(src, dst, g3).reshape(NC, p, d3)

    outp = pl.pallas_call(
        _tc4_body,
        out_shape=_f32((p, d3))[0],
    )(a3, g3, dinv, b3p)

    return outp[:n, :2]
